# baseline (speedup 1.0000x reference)
import jax
import jax.numpy as jnp
from jax import lax
from jax.experimental import pallas as pl
from jax.experimental.pallas import tpu as pltpu

_N_TOKENS = 16384
_NUM_CODES = 1024
_DIM = 64
_TILE = 2048
_GRID = _N_TOKENS // _TILE


def _vq_body(zt_ref, et_ref, qt_ref, idx_ref, loss_ref):
    zt = zt_ref[...]
    et = et_ref[...]
    zsq = jnp.sum(zt * zt, axis=0, keepdims=True)
    esq = jnp.sum(et * et, axis=0)[:, None]
    mmt = lax.dot_general(
        et, zt, (((0,), (0,)), ((), ())), preferred_element_type=jnp.float32
    )
    d = (zsq - 2.0 * mmt) + esq
    mins = jnp.min(d, axis=0, keepdims=True)
    ii = lax.broadcasted_iota(jnp.int32, (_NUM_CODES, _TILE), 0)
    idx = jnp.min(jnp.where(d == mins, ii, _NUM_CODES), axis=0)
    idx_ref[...] = idx
    onehot = (ii == idx[None, :]).astype(jnp.float32)
    qt_ref[...] = lax.dot_general(
        et, onehot, (((1,), (0,)), ((), ())), preferred_element_type=jnp.float32
    )
    tile_sum = jnp.sum(mins)

    @pl.when(pl.program_id(0) == 0)
    def _():
        loss_ref[0, 0] = 0.0

    loss_ref[0, 0] += tile_sum


def kernel(z, embeddings):
    zt = z.T
    et = embeddings.T
    qt, idx, loss_acc = pl.pallas_call(
        _vq_body,
        grid=(_GRID,),
        in_specs=[
            pl.BlockSpec((_DIM, _TILE), lambda i: (0, i)),
            pl.BlockSpec((_DIM, _NUM_CODES), lambda i: (0, 0)),
        ],
        out_specs=(
            pl.BlockSpec((_DIM, _TILE), lambda i: (0, i)),
            pl.BlockSpec((_TILE,), lambda i: (i,)),
            pl.BlockSpec(memory_space=pltpu.SMEM),
        ),
        out_shape=(
            jax.ShapeDtypeStruct((_DIM, _N_TOKENS), jnp.float32),
            jax.ShapeDtypeStruct((_N_TOKENS,), jnp.int32),
            jax.ShapeDtypeStruct((1, 1), jnp.float32),
        ),
        compiler_params=pltpu.CompilerParams(
            dimension_semantics=("arbitrary",),
        ),
    )(zt, et)
    loss = loss_acc[0, 0] / (_N_TOKENS * _DIM)
    return qt.T, idx, loss


# speedup vs baseline: 1.0222x; 1.0222x over previous
import jax
import jax.numpy as jnp
from jax import lax
from jax.experimental import pallas as pl
from jax.experimental.pallas import tpu as pltpu

_N_TOKENS = 16384
_NUM_CODES = 1024
_DIM = 64
_TILE = 8192
_GRID = _N_TOKENS // _TILE


def _vq_body(zt_ref, et_ref, qt_ref, idx_ref, loss_ref):
    zt = zt_ref[...]
    et = et_ref[...]
    zsq = jnp.sum(zt * zt, axis=0, keepdims=True)
    esq = jnp.sum(et * et, axis=0)[:, None]
    mmt = lax.dot_general(
        et, zt, (((0,), (0,)), ((), ())), preferred_element_type=jnp.float32
    )
    d = (zsq - 2.0 * mmt) + esq
    mins = jnp.min(d, axis=0, keepdims=True)
    ii = lax.broadcasted_iota(jnp.int32, (_NUM_CODES, _TILE), 0)
    idx = jnp.min(jnp.where(d == mins, ii, _NUM_CODES), axis=0)
    idx_ref[...] = idx
    onehot = (ii == idx[None, :]).astype(jnp.float32)
    qt_ref[...] = lax.dot_general(
        et, onehot, (((1,), (0,)), ((), ())), preferred_element_type=jnp.float32
    )
    tile_sum = jnp.sum(mins)

    @pl.when(pl.program_id(0) == 0)
    def _():
        loss_ref[0, 0] = 0.0

    loss_ref[0, 0] += tile_sum


def kernel(z, embeddings):
    zt = z.T
    et = embeddings.T
    qt, idx, loss_acc = pl.pallas_call(
        _vq_body,
        grid=(_GRID,),
        in_specs=[
            pl.BlockSpec((_DIM, _TILE), lambda i: (0, i)),
            pl.BlockSpec((_DIM, _NUM_CODES), lambda i: (0, 0)),
        ],
        out_specs=(
            pl.BlockSpec((_DIM, _TILE), lambda i: (0, i)),
            pl.BlockSpec((_TILE,), lambda i: (i,)),
            pl.BlockSpec(memory_space=pltpu.SMEM),
        ),
        out_shape=(
            jax.ShapeDtypeStruct((_DIM, _N_TOKENS), jnp.float32),
            jax.ShapeDtypeStruct((_N_TOKENS,), jnp.int32),
            jax.ShapeDtypeStruct((1, 1), jnp.float32),
        ),
        compiler_params=pltpu.CompilerParams(
            dimension_semantics=("arbitrary",),
        ),
    )(zt, et)
    loss = loss_acc[0, 0] / (_N_TOKENS * _DIM)
    return qt.T, idx, loss


# speedup vs baseline: 1.0277x; 1.0054x over previous
import jax
import jax.numpy as jnp
from jax import lax
from jax.experimental import pallas as pl
from jax.experimental.pallas import tpu as pltpu

_N_TOKENS = 16384
_NUM_CODES = 1024
_DIM = 64
_TILE = 4096
_GRID = _N_TOKENS // _TILE


def _vq_body(zt_ref, et_ref, qt_ref, idx_ref, loss_ref):
    zt = zt_ref[...]
    et = et_ref[...]
    zsq = jnp.sum(zt * zt, axis=0, keepdims=True)
    esq = jnp.sum(et * et, axis=0)[:, None]
    mmt = lax.dot_general(
        et, zt, (((0,), (0,)), ((), ())), preferred_element_type=jnp.float32
    )
    d = (zsq - 2.0 * mmt) + esq
    mins = jnp.min(d, axis=0, keepdims=True)
    ii = lax.broadcasted_iota(jnp.int32, (_NUM_CODES, _TILE), 0)
    idx = jnp.min(jnp.where(d == mins, ii, _NUM_CODES), axis=0)
    idx_ref[...] = idx
    onehot = (ii == idx[None, :]).astype(jnp.float32)
    qt_ref[...] = lax.dot_general(
        et, onehot, (((1,), (0,)), ((), ())), preferred_element_type=jnp.float32
    )
    tile_sum = jnp.sum(mins)

    @pl.when(pl.program_id(0) == 0)
    def _():
        loss_ref[0, 0] = 0.0

    loss_ref[0, 0] += tile_sum


def kernel(z, embeddings):
    zt = z.T
    et = embeddings.T
    qt, idx, loss_acc = pl.pallas_call(
        _vq_body,
        grid=(_GRID,),
        in_specs=[
            pl.BlockSpec((_DIM, _TILE), lambda i: (0, i)),
            pl.BlockSpec((_DIM, _NUM_CODES), lambda i: (0, 0)),
        ],
        out_specs=(
            pl.BlockSpec((_DIM, _TILE), lambda i: (0, i)),
            pl.BlockSpec((_TILE,), lambda i: (i,)),
            pl.BlockSpec(memory_space=pltpu.SMEM),
        ),
        out_shape=(
            jax.ShapeDtypeStruct((_DIM, _N_TOKENS), jnp.float32),
            jax.ShapeDtypeStruct((_N_TOKENS,), jnp.int32),
            jax.ShapeDtypeStruct((1, 1), jnp.float32),
        ),
        compiler_params=pltpu.CompilerParams(
            dimension_semantics=("arbitrary",),
        ),
    )(zt, et)
    loss = loss_acc[0, 0] / (_N_TOKENS * _DIM)
    return qt.T, idx, loss
